# baseline (speedup 1.0000x reference)
import jax
import jax.numpy as jnp
from jax.experimental import pallas as pl
from jax.experimental.pallas import tpu as pltpu

B, H, S, D, Q = 16, 16, 2048, 128, 16
BH = B * H
CB = 8


def _body(pos_ref, c_ref, val_ref, out_ref):
    out_ref[...] = c_ref[...]
    for c in range(CB):
        for q in range(Q):
            p = pos_ref[q]
            out_ref[c, pl.ds(p, 1), :] = val_ref[c, pl.ds(q, 1), :]


def _copy_scatter(cache, val, input_pos):
    grid_spec = pltpu.PrefetchScalarGridSpec(
        num_scalar_prefetch=1,
        grid=(BH // CB,),
        in_specs=[
            pl.BlockSpec((CB, S, D), lambda i, pos: (i, 0, 0)),
            pl.BlockSpec((CB, Q, D), lambda i, pos: (i, 0, 0)),
        ],
        out_specs=pl.BlockSpec((CB, S, D), lambda i, pos: (i, 0, 0)),
    )
    return pl.pallas_call(
        _body,
        grid_spec=grid_spec,
        out_shape=jax.ShapeDtypeStruct((BH, S, D), jnp.float32),
        compiler_params=pltpu.CompilerParams(
            dimension_semantics=("arbitrary",),
        ),
    )(input_pos, cache, val)


def kernel(k_cache, v_cache, input_pos, k_val, v_val):
    k_out = _copy_scatter(k_cache.reshape(BH, S, D),
                          k_val.reshape(BH, Q, D), input_pos)
    v_out = _copy_scatter(v_cache.reshape(BH, S, D),
                          v_val.reshape(BH, Q, D), input_pos)
    return (k_out.reshape(B, H, S, D), v_out.reshape(B, H, S, D))
